# Initial kernel scaffold; baseline (speedup 1.0000x reference)
import jax
import jax.numpy as jnp
from jax.experimental import pallas as pl


def kernel(x, m2v_x, params, label_y, label_idx, edge_src_0, edge_dst_0, edge_src_1, edge_dst_1):
    raise NotImplementedError("write your pallas kernel here")


# speedup vs baseline: 12.5474x; 12.5474x over previous
import functools

import jax
import jax.numpy as jnp
from jax import lax
from jax.experimental import pallas as pl
from jax.experimental.pallas import tpu as pltpu
from jax.experimental.pallas import tpu_sc as plsc

N0, N1, N2 = 10000, 5000, 1000
E0, E1 = 320000, 80000
IN, HID, H, NC, L = 128, 512, 8, 153, 500
D = HID // H
M2V = 64

F32 = jnp.float32
I32 = jnp.int32


def _bn(xv, g, b):
    m = jnp.mean(xv, axis=0, keepdims=True)
    v = jnp.mean((xv - m) ** 2, axis=0, keepdims=True)
    return (xv - m) * lax.rsqrt(v + 1e-5) * g + b


def _elu(x):
    return jnp.where(x > 0, x, jnp.exp(jnp.minimum(x, 0.0)) - 1.0)


def _lrelu(x):
    return jnp.where(x >= 0, x, 0.2 * x)


def _k1a_body(x_ref, m2v_ref, wm2v_ref, bm2v_ref, table_ref, ly_ref, lidx_ref,
              lw1_ref, lb1_ref, lng_ref, lnb_ref, lw2_ref, lb2_ref,
              x2_out, le_s, lf_s, hh_s):
    x2_out[...] = x_ref[...] + jnp.dot(m2v_ref[...], wm2v_ref[...],
                                       preferred_element_type=F32) + bm2v_ref[...]

    def gather_body(i, _):
        gi = ly_ref[i]
        fi = lidx_ref[i]
        le_s[pl.ds(i, 1), :] = table_ref[pl.ds(gi, 1), :]
        lf_s[pl.ds(i, 1), :] = x2_out[pl.ds(fi, 1), :]
        return _
    lax.fori_loop(0, L, gather_body, None)

    hcat = jnp.concatenate([le_s[...], lf_s[...]], axis=1)
    hh = jax.nn.relu(_bn(jnp.dot(hcat, lw1_ref[...], preferred_element_type=F32)
                         + lb1_ref[...], lng_ref[...], lnb_ref[...]))
    hh_s[...] = jnp.dot(hh, lw2_ref[...], preferred_element_type=F32) + lb2_ref[...]

    def scatter_body(i, _):
        fi = lidx_ref[i]
        x2_out[pl.ds(fi, 1), :] = hh_s[pl.ds(i, 1), :]
        return _
    lax.fori_loop(0, L, scatter_body, None)


def _proj_body(x_ref, w_ref, wel_ref, wer_ref, h_out, el_out, er_out, c_out):
    n_dst = er_out.shape[0]
    h_out[...] = jnp.dot(x_ref[...], w_ref[...], preferred_element_type=F32)
    el = jnp.dot(x_ref[...], wel_ref[...], preferred_element_type=F32)
    er = jnp.dot(x_ref[:n_dst], wer_ref[...], preferred_element_type=F32)
    el_out[...] = el
    er_out[...] = er
    c_out[...] = _lrelu(jnp.max(el, axis=0, keepdims=True)
                        + jnp.max(er, axis=0, keepdims=True))


def _skip_body(x_ref, ws_ref, bs_ref, g_ref, b_ref, xs_out):
    xs_out[...] = _elu(_bn(jnp.dot(x_ref[...], ws_ref[...],
                                   preferred_element_type=F32) + bs_ref[...],
                           g_ref[...], b_ref[...]))


def _proj(xv, w, wel, wer, n_dst):
    n, hid = xv.shape[0], w.shape[1]
    return pl.pallas_call(
        _proj_body,
        out_shape=[
            jax.ShapeDtypeStruct((n, hid), F32),
            jax.ShapeDtypeStruct((n, H), F32),
            jax.ShapeDtypeStruct((n_dst, H), F32),
            jax.ShapeDtypeStruct((1, H), F32),
        ],
    )(xv, w, wel, wer)


def _skip(xv, ws, bs, g, b):
    n = xv.shape[0]
    return pl.pallas_call(
        _skip_body,
        out_shape=jax.ShapeDtypeStruct((n, HID), F32),
    )(xv, ws, bs.reshape(1, HID), g.reshape(1, HID), b.reshape(1, HID))


def _comb_body(agga_ref, aggb_ref, bias_ref, bng_ref, bnb_ref, xs_ref,
               wpa_ref, bpa_ref, png_ref, pnb_ref, y_out):
    rst = jnp.concatenate([agga_ref[...], aggb_ref[...]], axis=1) + bias_ref[...]
    g = _elu(_bn(rst, bng_ref[...], bnb_ref[...]))
    xs = xs_ref[...]
    lh = jnp.dot(g, wpa_ref[...], preferred_element_type=F32) + bpa_ref[...]
    ls = jnp.dot(xs, wpa_ref[...], preferred_element_type=F32) + bpa_ref[...]
    m = jnp.maximum(lh, ls)
    eh = jnp.exp(lh - m)
    es = jnp.exp(ls - m)
    y_out[...] = _bn((eh * g + es * xs) / (eh + es), png_ref[...], pnb_ref[...])


def _comb(agga, aggb, pp, xs):
    n = agga.shape[0]
    return pl.pallas_call(
        _comb_body,
        out_shape=jax.ShapeDtypeStruct((n, HID), F32),
    )(agga, aggb, pp["bias"].reshape(1, HID), pp["bn_g"].reshape(1, HID),
      pp["bn_b"].reshape(1, HID), xs, pp["Wpa"], pp["bpa"].reshape(1, 1),
      pp["pn_g"].reshape(1, HID), pp["pn_b"].reshape(1, HID))


def _head_body(y_ref, mw1_ref, mb1_ref, mng_ref, mnb_ref, mw2_ref, mb2_ref,
               out_ref):
    t = jax.nn.relu(_bn(jnp.dot(y_ref[...], mw1_ref[...], preferred_element_type=F32)
                        + mb1_ref[...], mng_ref[...], mnb_ref[...]))
    o = jnp.dot(t, mw2_ref[...], preferred_element_type=F32) + mb2_ref[...]
    o = o - jnp.max(o, axis=1, keepdims=True)
    eo = jnp.exp(o)
    out_ref[...] = eo / jnp.sum(eo, axis=1, keepdims=True)


def _make_edge_kernel(E, n_src, n_dst, R, C, scan_chunk):
    n_dst_pad = R * 16
    KA = 128
    KB = 64
    n_scan = E // scan_chunk
    assert E % scan_chunk == 0 and scan_chunk % 16 == 0

    mesh = plsc.VectorSubcoreMesh(core_axis_name="c", subcore_axis_name="s")

    def body(hs, elx, erx, cpad, srce, dste, agg,
             cidx, acc, den, svm, dvm,
             elidx, hidx, elbuf, elbufb, eebuf, abuf, rowbuf, erslab, cbuf, sem):
        iota = lax.iota(I32, 16)
        iota_d4 = lax.shift_right_logical(iota, 2)
        iota_m4 = iota & 3
        c = lax.axis_index("c")
        s = lax.axis_index("s")
        r0 = s * R

        pltpu.sync_copy(cpad.at[c], cbuf)
        cvec = plsc.load_gather(cbuf, [iota_m4])
        pltpu.sync_copy(erx.at[c, pl.ds(r0, R)], erslab)

        def zacc(r, _):
            for v in range(16):
                acc[r, pl.ds(v * 16, 16)] = jnp.zeros((16,), F32)
            return _
        lax.fori_loop(0, R, zacc, None)
        nden = (R * 4 + 15) // 16
        for j in range(nden):
            den[pl.ds(j * 16, 16)] = jnp.zeros((16,), F32)

        def scan_chunk_body(t, cnt):
            pltpu.sync_copy(srce.at[pl.ds(t * scan_chunk, scan_chunk)], svm)
            pltpu.sync_copy(dste.at[pl.ds(t * scan_chunk, scan_chunk)], dvm)

            def scan_vec(j, cnt):
                dv = dvm[pl.ds(j * 16, 16)]
                sv = svm[pl.ds(j * 16, 16)]
                dloc = dv - r0
                msk = (dloc >= 0) & (dloc < R)
                packed = sv * 512 + dloc
                plsc.store_compressed(cidx.at[pl.ds(cnt, 16)], packed, mask=msk)
                return jnp.minimum(cnt + jnp.sum(msk.astype(I32)), C)
            return lax.fori_loop(0, scan_chunk // 16, scan_vec, cnt)
        cnt = lax.fori_loop(0, n_scan, scan_chunk_body, jnp.int32(0))

        for j in range(KA // 16 + 1):
            cidx[pl.ds(cnt + j * 16, 16)] = jnp.zeros((16,), I32)

        def passa(t, _):
            base = t * KA

            def mkidx(j, _):
                pk = cidx[pl.ds(base + j * 16, 16)]
                sv = lax.shift_right_logical(pk, 9)
                elidx[pl.ds(j * 16, 16)] = sv * 2 + c
                return _
            lax.fori_loop(0, KA // 16, mkidx, None)

            pltpu.async_copy(elx.at[elidx], elbuf, sem).wait()

            def eevec(j2, _):
                e0 = j2 * 4
                dl16 = plsc.load_gather(cidx, [base + e0 + iota_d4]) & 511
                elv = plsc.load_gather(elbuf, [e0 + iota_d4, iota_m4])
                erv = plsc.load_gather(erslab, [dl16, iota_m4])
                e = elv + erv
                e = jnp.where(e >= 0, e, 0.2 * e)
                ee = jnp.exp(e - cvec)
                eebuf[pl.ds(j2 * 16, 16)] = ee
                for k in range(4):
                    dlr = plsc.load_gather(cidx, [jnp.full((16,), base + e0 + k, I32)]) & 511
                    val = plsc.load_gather(eebuf, [jnp.full((16,), (e0 + k) * 4, I32) + iota_m4])
                    mk = (iota < 4) & (base + e0 + k < cnt)
                    plsc.addupdate_scatter(den, [dlr * 4 + iota_m4], val, mask=mk)
                return _
            lax.fori_loop(0, KA * 4 // 16, eevec, None)
            return _
        lax.fori_loop(0, (cnt + KA - 1) // KA, passa, None)

        for j in range(nden):
            den_v = den[pl.ds(j * 16, 16)]
            den[pl.ds(j * 16, 16)] = 1.0 / den_v

        def passb(t, _):
            base = t * KB

            def mkidx(j, _):
                pk = cidx[pl.ds(base + j * 16, 16)]
                sv = lax.shift_right_logical(pk, 9)
                hidx[pl.ds(j * 16, 16)] = sv * 2 + c
                return _
            lax.fori_loop(0, KB // 16, mkidx, None)

            cp1 = pltpu.async_copy(elx.at[hidx], elbufb, sem)
            cp3 = pltpu.async_copy(hs.at[hidx], rowbuf, sem)
            cp1.wait()
            cp3.wait()

            def avec(j2, _):
                e0 = j2 * 4
                dl16 = plsc.load_gather(cidx, [base + e0 + iota_d4]) & 511
                elv = plsc.load_gather(elbufb, [e0 + iota_d4, iota_m4])
                erv = plsc.load_gather(erslab, [dl16, iota_m4])
                e = elv + erv
                e = jnp.where(e >= 0, e, 0.2 * e)
                ee = jnp.exp(e - cvec)
                rd = plsc.load_gather(den, [dl16 * 4 + iota_m4])
                abuf[pl.ds(j2 * 16, 16)] = ee * rd
                return _
            lax.fori_loop(0, KB * 4 // 16, avec, None)

            def edge(i, _):
                dlr = plsc.load_gather(cidx, [jnp.full((16,), base + i, I32)]) & 511
                valid = (base + i) < cnt
                mk = (iota >= 0) & valid
                av = [plsc.load_gather(abuf, [jnp.full((16,), i * 4 + h, I32)])
                      for h in range(4)]
                for v in range(16):
                    rv = rowbuf[i, pl.ds(v * 16, 16)]
                    prod = rv * av[v // 4]
                    plsc.addupdate_scatter(acc, [dlr, iota + v * 16], prod, mask=mk)
                return _
            lax.fori_loop(0, KB, edge, None)
            return _
        lax.fori_loop(0, (cnt + KB - 1) // KB, passb, None)

        pltpu.sync_copy(acc, agg.at[pl.ds(c * n_dst_pad + r0, R)])

    kern = pl.kernel(
        body,
        out_type=jax.ShapeDtypeStruct((2 * n_dst_pad, 256), F32),
        mesh=mesh,
        compiler_params=pltpu.CompilerParams(needs_layout_passes=False,
                                             use_tc_tiling_on_sc=False),
        scratch_types=[
            pltpu.VMEM((C + KA + 16,), I32),
            pltpu.VMEM((R, 256), F32),
            pltpu.VMEM(((R * 4 + 15) // 16 * 16,), F32),
            pltpu.VMEM((scan_chunk,), I32),
            pltpu.VMEM((scan_chunk,), I32),
            pltpu.VMEM((KA,), I32),
            pltpu.VMEM((KB,), I32),
            pltpu.VMEM((KA, 16), F32),
            pltpu.VMEM((KB, 16), F32),
            pltpu.VMEM((KA * 4,), F32),
            pltpu.VMEM((KB * 4,), F32),
            pltpu.VMEM((KB, 256), F32),
            pltpu.VMEM((R, 4), F32),
            pltpu.VMEM((16,), F32),
            pltpu.SemaphoreType.DMA,
        ],
    )
    return kern


_make_edge_kernel = functools.lru_cache(maxsize=None)(_make_edge_kernel)


def _run_edge(edge_kern, h_all, el_all, er_all, cb, src, dst, n_dst, R):
    n_dst_pad = R * 16
    hs = h_all.reshape(2 * h_all.shape[0], 256)
    elx = jnp.pad(el_all.reshape(2 * el_all.shape[0], 4), ((0, 0), (0, 12)))
    erx = jnp.pad(er_all.reshape(n_dst, 2, 4).transpose(1, 0, 2),
                  ((0, 0), (0, n_dst_pad - n_dst), (0, 0)))
    cpad = jnp.pad(cb.reshape(2, 4), ((0, 0), (0, 12)))
    agg = edge_kern(hs, elx, erx, cpad, src.astype(I32), dst.astype(I32))
    agg = agg.reshape(2, n_dst_pad, 256)[:, :n_dst, :]
    return agg[0], agg[1]


def kernel(x, m2v_x, params, label_y, label_idx,
           edge_src_0, edge_dst_0, edge_src_1, edge_dst_1):
    p = params
    p0, p1 = p["layer0"], p["layer1"]

    def _awei(pp):
        ael = (jnp.eye(H, dtype=F32)[:, None, :] * pp["al"][:, :, None]).reshape(HID, H)
        aer = (jnp.eye(H, dtype=F32)[:, None, :] * pp["ar"][:, :, None]).reshape(HID, H)
        return pp["W"] @ ael, pp["W"] @ aer

    wel0, wer0 = _awei(p0)
    wel1, wer1 = _awei(p1)

    vspec = pl.BlockSpec(memory_space=pltpu.VMEM)
    sspec = pl.BlockSpec(memory_space=pltpu.SMEM)

    x2 = pl.pallas_call(
        _k1a_body,
        out_shape=jax.ShapeDtypeStruct((N0, IN), F32),
        in_specs=[vspec] * 5 + [sspec] * 2 + [vspec] * 6,
        scratch_shapes=[
            pltpu.VMEM((L, IN), F32),
            pltpu.VMEM((L, IN), F32),
            pltpu.VMEM((L, IN), F32),
        ],
    )(x, m2v_x, p["Wm2v"], p["bm2v"].reshape(1, IN), p["label_table"],
      label_y.astype(I32), label_idx.astype(I32),
      p["lW1"], p["lb1"].reshape(1, HID), p["ln_g"].reshape(1, HID),
      p["ln_b"].reshape(1, HID), p["lW2"], p["lb2"].reshape(1, IN))

    h0, el0, er0, c0 = _proj(x2, p0["W"], wel0, wer0, N1)
    xs0 = _skip(x2[:N1], p0["Ws"], p0["bs"], p0["sn_g"], p0["sn_b"])

    agg0a, agg0b = _run_edge(_make_edge_kernel(E0, N0, N1, 320, 21504, 1600),
                             h0, el0, er0, c0, edge_src_0, edge_dst_0, N1, 320)

    y1 = _comb(agg0a, agg0b, p0, xs0)
    h1, el1, er1, c1 = _proj(y1, p1["W"], wel1, wer1, N2)
    xs1 = _skip(y1[:N2], p1["Ws"], p1["bs"], p1["sn_g"], p1["sn_b"])

    agg1a, agg1b = _run_edge(_make_edge_kernel(E1, N1, N2, 64, 6144, 1600),
                             h1, el1, er1, c1, edge_src_1, edge_dst_1, N2, 64)

    y2 = _comb(agg1a, agg1b, p1, xs1)

    out = pl.pallas_call(
        _head_body,
        out_shape=jax.ShapeDtypeStruct((N2, NC), F32),
    )(y2, p["mW1"], p["mb1"].reshape(1, HID), p["mn_g"].reshape(1, HID),
      p["mn_b"].reshape(1, HID), p["mW2"], p["mb2"].reshape(1, NC))
    return out


# speedup vs baseline: 14.5280x; 1.1579x over previous
import functools

import jax
import jax.numpy as jnp
from jax import lax
from jax.experimental import pallas as pl
from jax.experimental.pallas import tpu as pltpu
from jax.experimental.pallas import tpu_sc as plsc

N0, N1, N2 = 10000, 5000, 1000
E0, E1 = 320000, 80000
IN, HID, H, NC, L = 128, 512, 8, 153, 500
D = HID // H
M2V = 64

F32 = jnp.float32
I32 = jnp.int32


def _bn(xv, g, b):
    m = jnp.mean(xv, axis=0, keepdims=True)
    v = jnp.mean((xv - m) ** 2, axis=0, keepdims=True)
    return (xv - m) * lax.rsqrt(v + 1e-5) * g + b


def _elu(x):
    return jnp.where(x > 0, x, jnp.exp(jnp.minimum(x, 0.0)) - 1.0)


def _lrelu(x):
    return jnp.where(x >= 0, x, 0.2 * x)


def _k1a_body(x_ref, m2v_ref, wm2v_ref, bm2v_ref, table_ref, ly_ref, lidx_ref,
              lw1_ref, lb1_ref, lng_ref, lnb_ref, lw2_ref, lb2_ref,
              x2_out, le_s, lf_s, hh_s):
    x2_out[...] = x_ref[...] + jnp.dot(m2v_ref[...], wm2v_ref[...],
                                       preferred_element_type=F32) + bm2v_ref[...]

    def gather_body(i, _):
        gi = ly_ref[i]
        fi = lidx_ref[i]
        le_s[pl.ds(i, 1), :] = table_ref[pl.ds(gi, 1), :]
        lf_s[pl.ds(i, 1), :] = x2_out[pl.ds(fi, 1), :]
        return _
    lax.fori_loop(0, L, gather_body, None)

    hcat = jnp.concatenate([le_s[...], lf_s[...]], axis=1)
    hh = jax.nn.relu(_bn(jnp.dot(hcat, lw1_ref[...], preferred_element_type=F32)
                         + lb1_ref[...], lng_ref[...], lnb_ref[...]))
    hh_s[...] = jnp.dot(hh, lw2_ref[...], preferred_element_type=F32) + lb2_ref[...]

    def scatter_body(i, _):
        fi = lidx_ref[i]
        x2_out[pl.ds(fi, 1), :] = hh_s[pl.ds(i, 1), :]
        return _
    lax.fori_loop(0, L, scatter_body, None)


def _proj_body(x_ref, w_ref, wel_ref, wer_ref, h_out, el_out, er_out, c_out):
    n_dst = er_out.shape[0]
    h_out[...] = jnp.dot(x_ref[...], w_ref[...], preferred_element_type=F32)
    el = jnp.dot(x_ref[...], wel_ref[...], preferred_element_type=F32)
    er = jnp.dot(x_ref[:n_dst], wer_ref[...], preferred_element_type=F32)
    el_out[...] = el
    er_out[...] = er
    c_out[...] = _lrelu(jnp.max(el, axis=0, keepdims=True)
                        + jnp.max(er, axis=0, keepdims=True))


def _skip_body(x_ref, ws_ref, bs_ref, g_ref, b_ref, xs_out):
    xs_out[...] = _elu(_bn(jnp.dot(x_ref[...], ws_ref[...],
                                   preferred_element_type=F32) + bs_ref[...],
                           g_ref[...], b_ref[...]))


def _proj(xv, w, wel, wer, n_dst):
    n, hid = xv.shape[0], w.shape[1]
    return pl.pallas_call(
        _proj_body,
        out_shape=[
            jax.ShapeDtypeStruct((n, hid), F32),
            jax.ShapeDtypeStruct((n, H), F32),
            jax.ShapeDtypeStruct((n_dst, H), F32),
            jax.ShapeDtypeStruct((1, H), F32),
        ],
    )(xv, w, wel, wer)


def _skip(xv, ws, bs, g, b):
    n = xv.shape[0]
    return pl.pallas_call(
        _skip_body,
        out_shape=jax.ShapeDtypeStruct((n, HID), F32),
    )(xv, ws, bs.reshape(1, HID), g.reshape(1, HID), b.reshape(1, HID))


def _comb_body(agga_ref, aggb_ref, bias_ref, bng_ref, bnb_ref, xs_ref,
               wpa_ref, bpa_ref, png_ref, pnb_ref, y_out):
    rst = jnp.concatenate([agga_ref[...], aggb_ref[...]], axis=1) + bias_ref[...]
    g = _elu(_bn(rst, bng_ref[...], bnb_ref[...]))
    xs = xs_ref[...]
    lh = jnp.dot(g, wpa_ref[...], preferred_element_type=F32) + bpa_ref[...]
    ls = jnp.dot(xs, wpa_ref[...], preferred_element_type=F32) + bpa_ref[...]
    m = jnp.maximum(lh, ls)
    eh = jnp.exp(lh - m)
    es = jnp.exp(ls - m)
    y_out[...] = _bn((eh * g + es * xs) / (eh + es), png_ref[...], pnb_ref[...])


def _comb(agga, aggb, pp, xs):
    n = agga.shape[0]
    return pl.pallas_call(
        _comb_body,
        out_shape=jax.ShapeDtypeStruct((n, HID), F32),
    )(agga, aggb, pp["bias"].reshape(1, HID), pp["bn_g"].reshape(1, HID),
      pp["bn_b"].reshape(1, HID), xs, pp["Wpa"], pp["bpa"].reshape(1, 1),
      pp["pn_g"].reshape(1, HID), pp["pn_b"].reshape(1, HID))


def _head_body(y_ref, mw1_ref, mb1_ref, mng_ref, mnb_ref, mw2_ref, mb2_ref,
               out_ref):
    t = jax.nn.relu(_bn(jnp.dot(y_ref[...], mw1_ref[...], preferred_element_type=F32)
                        + mb1_ref[...], mng_ref[...], mnb_ref[...]))
    o = jnp.dot(t, mw2_ref[...], preferred_element_type=F32) + mb2_ref[...]
    o = o - jnp.max(o, axis=1, keepdims=True)
    eo = jnp.exp(o)
    out_ref[...] = eo / jnp.sum(eo, axis=1, keepdims=True)


def _make_edge_kernel(E, n_src, n_dst, R, C, scan_chunk):
    n_dst_pad = R * 16
    KA = 128
    KB = 32
    n_scan = E // scan_chunk
    assert E % scan_chunk == 0 and scan_chunk % 16 == 0

    mesh = plsc.VectorSubcoreMesh(core_axis_name="c", subcore_axis_name="s")

    def body(hs, elx, erx, cpad, srce, dste, agg,
             cidx, acc, den, svm, dvm,
             elidx, hidx0, hidx1, elbuf, elbufb0, elbufb1, eebuf, abuf,
             rowbuf0, rowbuf1, erslab, cbuf, sem, semb0, semb1):
        iota = lax.iota(I32, 16)
        iota_d4 = lax.shift_right_logical(iota, 2)
        iota_m4 = iota & 3
        c = lax.axis_index("c")
        s = lax.axis_index("s")
        r0 = s * R

        pltpu.sync_copy(cpad.at[c], cbuf)
        cvec = plsc.load_gather(cbuf, [iota_m4])
        pltpu.sync_copy(erx.at[c, pl.ds(r0, R)], erslab)

        def zacc(r, _):
            for v in range(16):
                acc[r, pl.ds(v * 16, 16)] = jnp.zeros((16,), F32)
            return _
        lax.fori_loop(0, R, zacc, None)
        nden = (R * 4 + 15) // 16
        for j in range(nden):
            den[pl.ds(j * 16, 16)] = jnp.zeros((16,), F32)

        def scan_chunk_body(t, cnt):
            pltpu.sync_copy(srce.at[pl.ds(t * scan_chunk, scan_chunk)], svm)
            pltpu.sync_copy(dste.at[pl.ds(t * scan_chunk, scan_chunk)], dvm)

            def scan_vec(j, cnt):
                dv = dvm[pl.ds(j * 16, 16)]
                sv = svm[pl.ds(j * 16, 16)]
                dloc = dv - r0
                msk = (dloc >= 0) & (dloc < R)
                packed = sv * 512 + dloc
                plsc.store_compressed(cidx.at[pl.ds(cnt, 16)], packed, mask=msk)
                return jnp.minimum(cnt + jnp.sum(msk.astype(I32)), C)
            return lax.fori_loop(0, scan_chunk // 16, scan_vec, cnt)
        cnt = lax.fori_loop(0, n_scan, scan_chunk_body, jnp.int32(0))

        for j in range(KA // 16 + 1):
            cidx[pl.ds(cnt + j * 16, 16)] = jnp.zeros((16,), I32)

        def passa(t, _):
            base = t * KA

            def mkidx(j, _):
                pk = cidx[pl.ds(base + j * 16, 16)]
                sv = lax.shift_right_logical(pk, 9)
                elidx[pl.ds(j * 16, 16)] = sv * 2 + c
                return _
            lax.fori_loop(0, KA // 16, mkidx, None)

            pltpu.async_copy(elx.at[elidx], elbuf, sem).wait()

            def eevec(j2, _):
                e0 = j2 * 4
                dl16 = plsc.load_gather(cidx, [base + e0 + iota_d4]) & 511
                elv = plsc.load_gather(elbuf, [e0 + iota_d4, iota_m4])
                erv = plsc.load_gather(erslab, [dl16, iota_m4])
                e = elv + erv
                e = jnp.where(e >= 0, e, 0.2 * e)
                ee = jnp.exp(e - cvec)
                eebuf[pl.ds(j2 * 16, 16)] = ee
                for k in range(4):
                    dlr = plsc.load_gather(cidx, [jnp.full((16,), base + e0 + k, I32)]) & 511
                    val = plsc.load_gather(eebuf, [jnp.full((16,), (e0 + k) * 4, I32) + iota_m4])
                    mk = (iota < 4) & (base + e0 + k < cnt)
                    plsc.addupdate_scatter(den, [dlr * 4 + iota_m4], val, mask=mk)
                return _
            lax.fori_loop(0, KA * 4 // 16, eevec, None)
            return _
        lax.fori_loop(0, (cnt + KA - 1) // KA, passa, None)

        for j in range(nden):
            den_v = den[pl.ds(j * 16, 16)]
            den[pl.ds(j * 16, 16)] = 1.0 / den_v

        nchb = (cnt + KB - 1) // KB

        def fire_b(t, hidx_i, elbufb_i, rowbuf_i, sem_i):
            def mkidx(j, _):
                pk = cidx[pl.ds(t * KB + j * 16, 16)]
                hidx_i[pl.ds(j * 16, 16)] = lax.shift_right_logical(pk, 9) * 2 + c
                return _
            lax.fori_loop(0, KB // 16, mkidx, None)
            pltpu.async_copy(elx.at[hidx_i], elbufb_i, sem_i)
            pltpu.async_copy(hs.at[hidx_i], rowbuf_i, sem_i)

        def wait_b(hidx_i, elbufb_i, rowbuf_i, sem_i):
            pltpu.make_async_copy(elx.at[hidx_i], elbufb_i, sem_i).wait()
            pltpu.make_async_copy(hs.at[hidx_i], rowbuf_i, sem_i).wait()

        def proc_b(t, elbufb_i, rowbuf_i):
            base = t * KB

            def avec(j2, _):
                e0 = j2 * 4
                dl16 = plsc.load_gather(cidx, [base + e0 + iota_d4]) & 511
                elv = plsc.load_gather(elbufb_i, [e0 + iota_d4, iota_m4])
                erv = plsc.load_gather(erslab, [dl16, iota_m4])
                e = elv + erv
                e = jnp.where(e >= 0, e, 0.2 * e)
                ee = jnp.exp(e - cvec)
                rd = plsc.load_gather(den, [dl16 * 4 + iota_m4])
                abuf[pl.ds(j2 * 16, 16)] = ee * rd
                return _
            lax.fori_loop(0, KB * 4 // 16, avec, None)

            def edge(i, _):
                dlr = plsc.load_gather(cidx, [jnp.full((16,), base + i, I32)]) & 511
                valid = (base + i) < cnt
                mk = (iota >= 0) & valid
                av = [plsc.load_gather(abuf, [jnp.full((16,), i * 4 + h, I32)])
                      for h in range(4)]
                for v in range(16):
                    rv = rowbuf_i[i, pl.ds(v * 16, 16)]
                    prod = rv * av[v // 4]
                    plsc.addupdate_scatter(acc, [dlr, iota + v * 16], prod, mask=mk)
                return _
            lax.fori_loop(0, KB, edge, None)

        set0 = (hidx0, elbufb0, rowbuf0, semb0)
        set1 = (hidx1, elbufb1, rowbuf1, semb1)
        fire_b(0, *set0)

        def pairbody(k, _):
            t0 = 2 * k
            t1 = 2 * k + 1

            @pl.when(t1 < nchb)
            def _f1():
                fire_b(t1, *set1)
            wait_b(*set0)
            proc_b(t0, set0[1], set0[2])

            @pl.when(t1 < nchb)
            def _p1():
                @pl.when(t1 + 1 < nchb)
                def _f2():
                    fire_b(t1 + 1, *set0)
                wait_b(*set1)
                proc_b(t1, set1[1], set1[2])
            return _
        lax.fori_loop(0, (nchb + 1) // 2, pairbody, None)

        pltpu.sync_copy(acc, agg.at[pl.ds(c * n_dst_pad + r0, R)])

    kern = pl.kernel(
        body,
        out_type=jax.ShapeDtypeStruct((2 * n_dst_pad, 256), F32),
        mesh=mesh,
        compiler_params=pltpu.CompilerParams(needs_layout_passes=False,
                                             use_tc_tiling_on_sc=False),
        scratch_types=[
            pltpu.VMEM((C + KA + 16,), I32),
            pltpu.VMEM((R, 256), F32),
            pltpu.VMEM(((R * 4 + 15) // 16 * 16,), F32),
            pltpu.VMEM((scan_chunk,), I32),
            pltpu.VMEM((scan_chunk,), I32),
            pltpu.VMEM((KA,), I32),
            pltpu.VMEM((KB,), I32),
            pltpu.VMEM((KB,), I32),
            pltpu.VMEM((KA, 16), F32),
            pltpu.VMEM((KB, 16), F32),
            pltpu.VMEM((KB, 16), F32),
            pltpu.VMEM((KA * 4,), F32),
            pltpu.VMEM((KB * 4,), F32),
            pltpu.VMEM((KB, 256), F32),
            pltpu.VMEM((KB, 256), F32),
            pltpu.VMEM((R, 4), F32),
            pltpu.VMEM((16,), F32),
            pltpu.SemaphoreType.DMA,
            pltpu.SemaphoreType.DMA,
            pltpu.SemaphoreType.DMA,
        ],
    )
    return kern


_make_edge_kernel = functools.lru_cache(maxsize=None)(_make_edge_kernel)


def _run_edge(edge_kern, h_all, el_all, er_all, cb, src, dst, n_dst, R):
    n_dst_pad = R * 16
    hs = h_all.reshape(2 * h_all.shape[0], 256)
    elx = jnp.pad(el_all.reshape(2 * el_all.shape[0], 4), ((0, 0), (0, 12)))
    erx = jnp.pad(er_all.reshape(n_dst, 2, 4).transpose(1, 0, 2),
                  ((0, 0), (0, n_dst_pad - n_dst), (0, 0)))
    cpad = jnp.pad(cb.reshape(2, 4), ((0, 0), (0, 12)))
    agg = edge_kern(hs, elx, erx, cpad, src.astype(I32), dst.astype(I32))
    agg = agg.reshape(2, n_dst_pad, 256)[:, :n_dst, :]
    return agg[0], agg[1]


def kernel(x, m2v_x, params, label_y, label_idx,
           edge_src_0, edge_dst_0, edge_src_1, edge_dst_1):
    p = params
    p0, p1 = p["layer0"], p["layer1"]

    def _awei(pp):
        ael = (jnp.eye(H, dtype=F32)[:, None, :] * pp["al"][:, :, None]).reshape(HID, H)
        aer = (jnp.eye(H, dtype=F32)[:, None, :] * pp["ar"][:, :, None]).reshape(HID, H)
        return pp["W"] @ ael, pp["W"] @ aer

    wel0, wer0 = _awei(p0)
    wel1, wer1 = _awei(p1)

    vspec = pl.BlockSpec(memory_space=pltpu.VMEM)
    sspec = pl.BlockSpec(memory_space=pltpu.SMEM)

    x2 = pl.pallas_call(
        _k1a_body,
        out_shape=jax.ShapeDtypeStruct((N0, IN), F32),
        in_specs=[vspec] * 5 + [sspec] * 2 + [vspec] * 6,
        scratch_shapes=[
            pltpu.VMEM((L, IN), F32),
            pltpu.VMEM((L, IN), F32),
            pltpu.VMEM((L, IN), F32),
        ],
    )(x, m2v_x, p["Wm2v"], p["bm2v"].reshape(1, IN), p["label_table"],
      label_y.astype(I32), label_idx.astype(I32),
      p["lW1"], p["lb1"].reshape(1, HID), p["ln_g"].reshape(1, HID),
      p["ln_b"].reshape(1, HID), p["lW2"], p["lb2"].reshape(1, IN))

    h0, el0, er0, c0 = _proj(x2, p0["W"], wel0, wer0, N1)
    xs0 = _skip(x2[:N1], p0["Ws"], p0["bs"], p0["sn_g"], p0["sn_b"])

    agg0a, agg0b = _run_edge(_make_edge_kernel(E0, N0, N1, 320, 21504, 1600),
                             h0, el0, er0, c0, edge_src_0, edge_dst_0, N1, 320)

    y1 = _comb(agg0a, agg0b, p0, xs0)
    h1, el1, er1, c1 = _proj(y1, p1["W"], wel1, wer1, N2)
    xs1 = _skip(y1[:N2], p1["Ws"], p1["bs"], p1["sn_g"], p1["sn_b"])

    agg1a, agg1b = _run_edge(_make_edge_kernel(E1, N1, N2, 64, 6144, 1600),
                             h1, el1, er1, c1, edge_src_1, edge_dst_1, N2, 64)

    y2 = _comb(agg1a, agg1b, p1, xs1)

    out = pl.pallas_call(
        _head_body,
        out_shape=jax.ShapeDtypeStruct((N2, NC), F32),
    )(y2, p["mW1"], p["mb1"].reshape(1, HID), p["mn_g"].reshape(1, HID),
      p["mn_b"].reshape(1, HID), p["mW2"], p["mb2"].reshape(1, NC))
    return out
